# baseline (speedup 1.0000x reference)
import functools

import jax
import jax.numpy as jnp
from jax import lax
from jax.experimental import pallas as pl
from jax.experimental.pallas import tpu as pltpu
from jax.experimental.pallas import tpu_sc as plsc

CHUNK = 128
NBUF = 2
PADD = 128


@functools.cache
def _build(B: int):
    info = plsc.get_sparse_core_info()
    NC, NS, L = info.num_cores, info.num_subcores, info.num_lanes
    NW = NC * NS
    assert B % (NW * CHUNK) == 0 and L == 16
    b_per_w = B // NW
    n_chunks = b_per_w // CHUNK
    assert n_chunks % NBUF == 0 and n_chunks // NBUF >= 2
    n_outer = n_chunks // NBUF

    mesh = plsc.VectorSubcoreMesh(core_axis_name="c", subcore_axis_name="s")

    @functools.partial(
        pl.kernel,
        mesh=mesh,
        compiler_params=pltpu.CompilerParams(use_tc_tiling_on_sc=True),
        out_type=jax.ShapeDtypeStruct((B * PADD // 2,), jnp.float32),
        scratch_types=[
            pltpu.VMEM((n_chunks * CHUNK,), jnp.int32),
            pltpu.VMEM((NBUF * CHUNK,), jnp.int32),
            pltpu.VMEM((NBUF, CHUNK, PADD), jnp.float32),
            pltpu.VMEM((NBUF * CHUNK * PADD // 2,), jnp.float32),
            pltpu.SemaphoreType.DMA((NBUF,)),
            pltpu.SemaphoreType.DMA((NBUF,)),
        ],
    )
    def gather_kernel(
        table_hbm, idx_hbm, out_hbm, idx_v, idx2_v, pairs_v, comp_v, gsem, ssem
    ):
        wid = lax.axis_index("s") * NC + lax.axis_index("c")
        base = pl.multiple_of(wid * b_per_w, 1024)

        pltpu.sync_copy(idx_hbm.at[pl.ds(base, b_per_w)], idx_v)

        def gather_copy(b):
            return pltpu.make_async_copy(
                table_hbm.at[idx2_v.at[pl.ds(b * CHUNK, CHUNK)]],
                pairs_v.at[b],
                gsem.at[b],
            )

        HALF = CHUNK * PADD // 2

        def store_copy(c, b):
            off = pl.multiple_of((base + c * CHUNK) * (PADD // 2), 8192)
            return pltpu.make_async_copy(
                comp_v.at[pl.ds(b * HALF, HALF)],
                out_hbm.at[pl.ds(off, HALF)],
                ssem.at[b],
            )

        def prep_indices(c, b):
            coff = pl.multiple_of(c * CHUNK, CHUNK)
            for g in range(CHUNK // L):
                v = idx_v[pl.ds(coff + g * L, L)]
                idx2_v[pl.ds(b * CHUNK + g * L, L)] = lax.shift_right_logical(v, 1)

        def compact(c, b):
            coff = pl.multiple_of(c * CHUNK, CHUNK)
            for g in range(CHUNK // L):
                vg = idx_v[pl.ds(coff + g * L, L)] & 1
                for l in range(L):
                    r = g * L + l
                    hoff = pl.multiple_of(vg[l] * 64, 64)
                    dst0 = b * HALF + r * 64
                    for j in range(4):
                        comp_v[pl.ds(dst0 + j * L, L)] = pairs_v[
                            b, r, pl.ds(hoff + j * L, L)
                        ]

        for b in range(NBUF):
            prep_indices(b, b)
            gather_copy(b).start()

        def outer(g, carry):
            c0 = g * NBUF
            for b in range(NBUF):
                gather_copy(b).wait()
                compact(c0 + b, b)
                store_copy(c0 + b, b).start()
            for b in range(NBUF):
                store_copy(c0 + b, b).wait()
                prep_indices(c0 + b + NBUF, b)
                gather_copy(b).start()
            return carry

        lax.fori_loop(0, n_outer - 1, outer, 0)

        c0 = (n_outer - 1) * NBUF
        for b in range(NBUF):
            gather_copy(b).wait()
            compact(c0 + b, b)
            store_copy(c0 + b, b).start()
        for b in range(NBUF):
            store_copy(c0 + b, b).wait()

    return gather_kernel


def kernel(x, embeddings):
    B = x.shape[0] * x.shape[1]
    D = embeddings.shape[1]
    idx = x.reshape(B).astype(jnp.int32)
    table = embeddings.reshape(embeddings.shape[0] // 2, 2 * D)
    out = _build(B)(table, idx)
    return out.reshape(x.shape[0], x.shape[1], D)


# speedup vs baseline: 1.4252x; 1.4252x over previous
import functools

import jax
import jax.numpy as jnp
from jax import lax
from jax.experimental import pallas as pl
from jax.experimental.pallas import tpu as pltpu
from jax.experimental.pallas import tpu_sc as plsc

CHUNK = 128
NBUF = 8


@functools.cache
def _build(B: int, D: int):
    info = plsc.get_sparse_core_info()
    NC, NS = info.num_cores, info.num_subcores
    NW = NC * NS
    assert B % (NW * CHUNK) == 0
    b_per_w = B // NW
    n_chunks = b_per_w // CHUNK
    assert n_chunks % NBUF == 0 and n_chunks // NBUF >= 2
    n_outer = n_chunks // NBUF

    mesh = plsc.VectorSubcoreMesh(core_axis_name="c", subcore_axis_name="s")

    @functools.partial(
        pl.kernel,
        mesh=mesh,
        compiler_params=pltpu.CompilerParams(use_tc_tiling_on_sc=False),
        out_type=jax.ShapeDtypeStruct((B, D), jnp.float32),
        scratch_types=[
            pltpu.VMEM((n_chunks, CHUNK), jnp.int32),
            pltpu.VMEM((NBUF, CHUNK, D), jnp.float32),
            pltpu.SemaphoreType.DMA((NBUF,)),
            pltpu.SemaphoreType.DMA((NBUF,)),
        ],
    )
    def gather_kernel(table_hbm, idx_hbm, out_hbm, idx_v, rows_v, gsem, ssem):
        wid = lax.axis_index("s") * NC + lax.axis_index("c")
        base = wid * b_per_w

        pltpu.sync_copy(idx_hbm.at[wid], idx_v)

        def gather_copy(c, b):
            return pltpu.make_async_copy(
                table_hbm.at[idx_v.at[c]], rows_v.at[b], gsem.at[b]
            )

        def store_copy(c, b):
            return pltpu.make_async_copy(
                rows_v.at[b], out_hbm.at[pl.ds(base + c * CHUNK, CHUNK)], ssem.at[b]
            )

        for b in range(NBUF):
            gather_copy(b, b).start()

        def outer(g, carry):
            c0 = g * NBUF
            for b in range(NBUF):
                gather_copy(c0 + b, b).wait()
                store_copy(c0 + b, b).start()
            for b in range(NBUF):
                store_copy(c0 + b, b).wait()
                gather_copy(c0 + b + NBUF, b).start()
            return carry

        lax.fori_loop(0, n_outer - 1, outer, 0)

        c0 = (n_outer - 1) * NBUF
        for b in range(NBUF):
            gather_copy(c0 + b, b).wait()
            store_copy(c0 + b, b).start()
        for b in range(NBUF):
            store_copy(c0 + b, b).wait()

    return gather_kernel


def kernel(x, embeddings):
    B = x.shape[0] * x.shape[1]
    D = embeddings.shape[1]
    info = plsc.get_sparse_core_info()
    NW = info.num_cores * info.num_subcores
    idx = x.reshape(NW, (B // NW) // CHUNK, CHUNK).astype(jnp.int32)
    out = _build(B, D)(embeddings, idx)
    return out.reshape(x.shape[0], x.shape[1], D)
